# Initial kernel scaffold; baseline (speedup 1.0000x reference)
import jax
import jax.numpy as jnp
from jax.experimental import pallas as pl


def kernel(x, edge_attr, edge_index, Wn1, bn1, We1, be1, Wm1, bm1, Wn2, bn2, We2, be2, Wm2, bm2):
    raise NotImplementedError("write your pallas kernel here")


# speedup vs baseline: 2.2131x; 2.2131x over previous
import functools

import jax
import jax.numpy as jnp
from jax import lax
from jax.experimental import pallas as pl
from jax.experimental.pallas import tpu as pltpu
from jax.experimental.pallas import tpu_sc as plsc

N = 10000
E = 160000
DIN = 128
DE = 16
DH = 128
DL = 64

NC = 2
NS = 16
NW = NC * NS
CK1 = 40
CK2 = 40
EPW = E // NW
NPAD = 10240
RPT = NPAD // NS

_SC_MESH = dict(core_axis_name="c", subcore_axis_name="s",
                num_cores=NC, num_subcores=NS)


def _zero_vmem_2d(ref, rows, cols):
    zv = jnp.zeros((16,), jnp.float32)

    def body(i, _):
        for j in range(cols // 16):
            ref[i, pl.ds(j * 16, 16)] = zv
        return 0

    lax.fori_loop(0, rows, body, 0, unroll=False)


def _copy_rows(src_at, dst_at, chunk):
    for k in range(RPT // chunk):
        pltpu.sync_copy(src_at(k * chunk, chunk), dst_at(k * chunk, chunk))


def _sc_edge_pass1(x, src_idx, dst_idx):
    mesh = plsc.VectorSubcoreMesh(**_SC_MESH)

    @functools.partial(
        pl.kernel,
        out_type=jax.ShapeDtypeStruct((NC, NPAD, DIN), jnp.float32),
        mesh=mesh,
        scratch_types=[
            pltpu.VMEM_SHARED((NPAD, DIN), jnp.float32),
            pltpu.VMEM((CK1, DIN), jnp.float32),
            pltpu.VMEM((CK1,), jnp.int32),
            pltpu.VMEM((CK1,), jnp.int32),
            pltpu.SemaphoreType.DMA,
        ],
    )
    def run(x_hbm, si_hbm, di_hbm, xs_out,
            acc_x, rows_v, src_v, dst_v, sem):
        c = lax.axis_index("c")
        s = lax.axis_index("s")
        wid = s * NC + c

        _zero_vmem_2d(rows_v, CK1, DIN)

        r0 = s * RPT
        _copy_rows(lambda o, z: rows_v.at[pl.ds(0, z), :],
                   lambda o, z: acc_x.at[pl.ds(r0 + o, z), :], CK1)
        plsc.subcore_barrier()

        e0 = wid * EPW

        def chunk_body(i, _):
            base = e0 + i * CK1
            pltpu.sync_copy(si_hbm.at[pl.ds(base, CK1)], src_v)
            pltpu.sync_copy(di_hbm.at[pl.ds(base, CK1)], dst_v)
            pltpu.async_copy(x_hbm.at[src_v], rows_v, sem).wait()
            pltpu.sync_copy(rows_v, acc_x.at[dst_v], add=True)
            return 0

        lax.fori_loop(0, EPW // CK1, chunk_body, 0, unroll=False)
        plsc.subcore_barrier()

        _copy_rows(lambda o, z: acc_x.at[pl.ds(r0 + o, z), :],
                   lambda o, z: xs_out.at[c, pl.ds(r0 + o, z), :], CK1)

    return run(x, src_idx, dst_idx)


CKE = 40


def _sc_edge_passE(edge_attr, dst_idx):
    mesh = plsc.VectorSubcoreMesh(**_SC_MESH)

    @functools.partial(
        pl.kernel,
        out_type=jax.ShapeDtypeStruct((NC, NPAD, DIN), jnp.float32),
        mesh=mesh,
        scratch_types=[
            pltpu.VMEM_SHARED((NPAD, DIN), jnp.float32),
            pltpu.VMEM((CKE, DIN), jnp.float32),
            pltpu.VMEM((CKE, DE), jnp.float32),
            pltpu.VMEM((CKE,), jnp.int32),
        ],
    )
    def run(ea_hbm, di_hbm, q_out, acc_q, wide_v, eav_v, dst_v):
        c = lax.axis_index("c")
        s = lax.axis_index("s")
        wid = s * NC + c

        _zero_vmem_2d(wide_v, CKE, DIN)

        r0 = s * RPT
        _copy_rows(lambda o, z: wide_v.at[pl.ds(0, z), :],
                   lambda o, z: acc_q.at[pl.ds(r0 + o, z), :], CKE)

        one = jnp.full((16,), 1.0, jnp.float32)

        def initrow(i, _):
            wide_v[i, pl.ds(DE, 16)] = one
            return 0

        lax.fori_loop(0, CKE, initrow, 0, unroll=False)
        plsc.subcore_barrier()

        e0 = wid * EPW

        def chunk_body(i, _):
            base = e0 + i * CKE
            pltpu.sync_copy(di_hbm.at[pl.ds(base, CKE)], dst_v)
            pltpu.sync_copy(ea_hbm.at[pl.ds(base, CKE), :], eav_v)

            def repack(e, _):
                wide_v[e, pl.ds(0, DE)] = eav_v[e, pl.ds(0, DE)]
                return 0

            lax.fori_loop(0, CKE, repack, 0, unroll=False)
            pltpu.sync_copy(wide_v, acc_q.at[dst_v], add=True)
            return 0

        lax.fori_loop(0, EPW // CKE, chunk_body, 0, unroll=False)
        plsc.subcore_barrier()

        _copy_rows(lambda o, z: acc_q.at[pl.ds(r0 + o, z), :],
                   lambda o, z: q_out.at[c, pl.ds(r0 + o, z), :], CKE)

    return run(edge_attr, dst_idx)


def _sc_edge_pass2(h1, src_idx, dst_idx):
    mesh = plsc.VectorSubcoreMesh(**_SC_MESH)

    @functools.partial(
        pl.kernel,
        out_type=jax.ShapeDtypeStruct((NC, NPAD, DH), jnp.float32),
        mesh=mesh,
        scratch_types=[
            pltpu.VMEM_SHARED((NPAD, DH), jnp.float32),
            pltpu.VMEM((CK2, DH), jnp.float32),
            pltpu.VMEM((CK2,), jnp.int32),
            pltpu.VMEM((CK2,), jnp.int32),
            pltpu.SemaphoreType.DMA,
        ],
    )
    def run(h_hbm, si_hbm, di_hbm, s2_out, acc, rows_v, src_v, dst_v, sem):
        c = lax.axis_index("c")
        s = lax.axis_index("s")
        wid = s * NC + c

        _zero_vmem_2d(rows_v, CK2, DH)
        r0 = s * RPT
        _copy_rows(lambda o, z: rows_v.at[pl.ds(0, z), :],
                   lambda o, z: acc.at[pl.ds(r0 + o, z), :], CK2)
        plsc.subcore_barrier()

        e0 = wid * EPW

        def chunk_body(i, _):
            base = e0 + i * CK2
            pltpu.sync_copy(si_hbm.at[pl.ds(base, CK2)], src_v)
            pltpu.sync_copy(di_hbm.at[pl.ds(base, CK2)], dst_v)
            pltpu.async_copy(h_hbm.at[src_v], rows_v, sem).wait()
            pltpu.sync_copy(rows_v, acc.at[dst_v], add=True)
            return 0

        lax.fori_loop(0, EPW // CK2, chunk_body, 0, unroll=False)
        plsc.subcore_barrier()

        _copy_rows(lambda o, z: acc.at[pl.ds(r0 + o, z), :],
                   lambda o, z: s2_out.at[c, pl.ds(r0 + o, z), :], CK2)

    return run(h1, src_idx, dst_idx)


_BM1 = 1000


def _tc_layer1(xs_p, q_p, Wn1, Wm1, We1, bn1, be1, bm1):
    def body(xs_ref, q_ref, wn1_ref, wm1_ref, we1_ref, bn1_ref,
             be1_ref, bm1_ref, out_ref):
        xs = xs_ref[0] + xs_ref[1]
        q = q_ref[0] + q_ref[1]
        ea = q[:, 0:DE]
        dg = q[:, DE:DE + 1]
        wm1 = wm1_ref[...]
        wt = wn1_ref[...] @ wm1
        we = we1_ref[...] @ wm1
        bt = (bn1_ref[...] + be1_ref[...]) @ wm1 + bm1_ref[...]
        sv = xs @ wt + ea @ we + dg * bt
        out_ref[...] = jnp.maximum(sv / jnp.clip(dg, 1.0, None), 0.0)

    grid = (N // _BM1,)
    return pl.pallas_call(
        body,
        grid=grid,
        in_specs=[
            pl.BlockSpec((NC, _BM1, DIN), lambda i: (0, i, 0)),
            pl.BlockSpec((NC, _BM1, DIN), lambda i: (0, i, 0)),
            pl.BlockSpec((DIN, DH), lambda i: (0, 0)),
            pl.BlockSpec((DH, DH), lambda i: (0, 0)),
            pl.BlockSpec((DE, DH), lambda i: (0, 0)),
            pl.BlockSpec((1, DH), lambda i: (0, 0)),
            pl.BlockSpec((1, DH), lambda i: (0, 0)),
            pl.BlockSpec((1, DH), lambda i: (0, 0)),
        ],
        out_specs=pl.BlockSpec((_BM1, DH), lambda i: (i, 0)),
        out_shape=jax.ShapeDtypeStruct((N, DH), jnp.float32),
    )(xs_p, q_p, Wn1, Wm1, We1, bn1, be1, bm1)


def _tc_z(s2_p, q_p, Wn2, bn2, We2, be2, Wm2, bm2):
    def body(s2_ref, q_ref, wn2_ref, bn2_ref, we2_ref, be2_ref,
             wm2_ref, bm2_ref, z_ref):
        s2 = s2_ref[0] + s2_ref[1]
        q = q_ref[0] + q_ref[1]
        ea = q[:, 0:DE]
        dg = q[:, DE:DE + 1]
        wm2 = wm2_ref[...]
        wn = wn2_ref[...] @ wm2
        we = we2_ref[...] @ wm2
        bt = (bn2_ref[...] + be2_ref[...]) @ wm2 + bm2_ref[...]
        sv = s2 @ wn + ea @ we + dg * bt
        z_ref[...] = sv / jnp.clip(dg, 1.0, None)

    grid = (N // _BM1,)
    return pl.pallas_call(
        body,
        grid=grid,
        in_specs=[
            pl.BlockSpec((NC, _BM1, DH), lambda i: (0, i, 0)),
            pl.BlockSpec((NC, _BM1, DIN), lambda i: (0, i, 0)),
            pl.BlockSpec((DH, DL), lambda i: (0, 0)),
            pl.BlockSpec((1, DL), lambda i: (0, 0)),
            pl.BlockSpec((DE, DL), lambda i: (0, 0)),
            pl.BlockSpec((1, DL), lambda i: (0, 0)),
            pl.BlockSpec((DL, DL), lambda i: (0, 0)),
            pl.BlockSpec((1, DL), lambda i: (0, 0)),
        ],
        out_specs=pl.BlockSpec((_BM1, DL), lambda i: (i, 0)),
        out_shape=jax.ShapeDtypeStruct((N, DL), jnp.float32),
    )(s2_p, q_p, Wn2, bn2, We2, be2, Wm2, bm2)


_BMA = 400


def _tc_adj(z):
    def body(zr_ref, zc_ref, out_ref):
        g = lax.dot_general(zr_ref[...], zc_ref[...],
                            (((1,), (1,)), ((), ())),
                            preferred_element_type=jnp.float32)
        out_ref[...] = jax.nn.sigmoid(g)

    grid = (N // _BMA,)
    return pl.pallas_call(
        body,
        grid=grid,
        in_specs=[
            pl.BlockSpec((_BMA, DL), lambda i: (i, 0)),
            pl.BlockSpec((N, DL), lambda i: (0, 0)),
        ],
        out_specs=pl.BlockSpec((_BMA, N), lambda i: (i, 0)),
        out_shape=jax.ShapeDtypeStruct((N, N), jnp.float32),
    )(z, z)


def kernel(x, edge_attr, edge_index, Wn1, bn1, We1, be1, Wm1, bm1,
           Wn2, bn2, We2, be2, Wm2, bm2):
    src_idx = edge_index[0]
    dst_idx = edge_index[1]
    q_p = _sc_edge_passE(edge_attr, dst_idx)
    xs_p = _sc_edge_pass1(x, src_idx, dst_idx)
    h1 = _tc_layer1(xs_p, q_p, Wn1, Wm1, We1,
                    bn1.reshape(1, DH), be1.reshape(1, DH),
                    bm1.reshape(1, DH))
    s2_p = _sc_edge_pass2(h1, src_idx, dst_idx)
    z = _tc_z(s2_p, q_p, Wn2, bn2.reshape(1, DL), We2,
              be2.reshape(1, DL), Wm2, bm2.reshape(1, DL))
    adj = _tc_adj(z)
    return (z, adj)


# speedup vs baseline: 4.2077x; 1.9013x over previous
import functools

import jax
import jax.numpy as jnp
from jax import lax
from jax.experimental import pallas as pl
from jax.experimental.pallas import tpu as pltpu
from jax.experimental.pallas import tpu_sc as plsc

N = 10000
E = 160000
DIN = 128
DE = 16
DH = 128
DL = 64

NC = 2
NS = 16
NW = NC * NS
CK1 = 40
EPW = E // NW
NPAD = 10240
RPT = NPAD // NS

_SC_MESH = dict(core_axis_name="c", subcore_axis_name="s",
                num_cores=NC, num_subcores=NS)


def _zero_vmem_2d(ref, rows, cols):
    zv = jnp.zeros((16,), jnp.float32)

    def body(i, _):
        for j in range(cols // 16):
            ref[i, pl.ds(j * 16, 16)] = zv
        return 0

    lax.fori_loop(0, rows, body, 0, unroll=False)


def _copy_rows(src_at, dst_at, chunk):
    for k in range(RPT // chunk):
        pltpu.sync_copy(src_at(k * chunk, chunk), dst_at(k * chunk, chunk))


NCHUNK = EPW // CK1


def _sc_gather_pass(table, src_idx, dst3, width):
    mesh = plsc.VectorSubcoreMesh(**_SC_MESH)

    @functools.partial(
        pl.kernel,
        out_type=jax.ShapeDtypeStruct((NC, NPAD, width), jnp.float32),
        mesh=mesh,
        scratch_types=[
            pltpu.VMEM_SHARED((NPAD, width), jnp.float32),
            pltpu.VMEM((CK1, width), jnp.float32),
            pltpu.VMEM((CK1, width), jnp.float32),
            pltpu.VMEM((EPW,), jnp.int32),
            pltpu.VMEM((NCHUNK, CK1), jnp.int32),
            pltpu.SemaphoreType.DMA,
        ],
    )
    def run(t_hbm, si_hbm, d2_hbm, acc_out,
            acc, rows0, rows1, srcall, dstall, sem):
        c = lax.axis_index("c")
        s = lax.axis_index("s")
        wid = s * NC + c

        _zero_vmem_2d(rows0, CK1, width)

        r0 = s * RPT
        _copy_rows(lambda o, z: rows0.at[pl.ds(0, z), :],
                   lambda o, z: acc.at[pl.ds(r0 + o, z), :], CK1)

        pltpu.sync_copy(si_hbm.at[pl.ds(wid * EPW, EPW)], srcall)
        pltpu.sync_copy(d2_hbm.at[wid], dstall)
        plsc.subcore_barrier()

        def start(i, buf):
            pltpu.async_copy(t_hbm.at[srcall.at[pl.ds(i * CK1, CK1)]],
                             buf, sem)

        def drain(buf):
            pltpu.make_async_copy(t_hbm.at[srcall.at[pl.ds(0, CK1)]],
                                  buf, sem).wait()

        start(0, rows0)

        def pair_body(g, _):
            i = 2 * g
            start(i + 1, rows1)
            drain(rows0)
            pltpu.sync_copy(rows0, acc.at[dstall.at[i]], add=True)
            start(i + 2, rows0)
            drain(rows1)
            pltpu.sync_copy(rows1, acc.at[dstall.at[i + 1]], add=True)
            return 0

        lax.fori_loop(0, NCHUNK // 2, pair_body, 0, unroll=False)
        drain(rows0)
        pltpu.sync_copy(rows0, acc.at[dstall.at[NCHUNK - 1]], add=True)
        plsc.subcore_barrier()

        _copy_rows(lambda o, z: acc.at[pl.ds(r0 + o, z), :],
                   lambda o, z: acc_out.at[c, pl.ds(r0 + o, z), :], CK1)

    return run(table, src_idx, dst3)


CKE = 40


def _sc_edge_passE(edge_attr, dst3):
    mesh = plsc.VectorSubcoreMesh(**_SC_MESH)

    @functools.partial(
        pl.kernel,
        out_type=jax.ShapeDtypeStruct((NC, NPAD, DIN), jnp.float32),
        mesh=mesh,
        scratch_types=[
            pltpu.VMEM_SHARED((NPAD, DIN), jnp.float32),
            pltpu.VMEM((CKE, DIN), jnp.float32),
            pltpu.VMEM((CKE, DE), jnp.float32),
            pltpu.VMEM((CKE, DE), jnp.float32),
            pltpu.VMEM((NCHUNK, CKE), jnp.int32),
            pltpu.SemaphoreType.DMA,
        ],
    )
    def run(ea_hbm, d2_hbm, q_out, acc_q, wide_v, eav0, eav1, dstall, sem):
        c = lax.axis_index("c")
        s = lax.axis_index("s")
        wid = s * NC + c

        _zero_vmem_2d(wide_v, CKE, DIN)

        r0 = s * RPT
        _copy_rows(lambda o, z: wide_v.at[pl.ds(0, z), :],
                   lambda o, z: acc_q.at[pl.ds(r0 + o, z), :], CKE)

        one = jnp.full((16,), 1.0, jnp.float32)

        def initrow(i, _):
            wide_v[i, pl.ds(DE, 16)] = one
            return 0

        lax.fori_loop(0, CKE, initrow, 0, unroll=False)
        pltpu.sync_copy(d2_hbm.at[wid], dstall)
        plsc.subcore_barrier()

        e0 = wid * EPW

        def start(i, buf):
            pltpu.async_copy(ea_hbm.at[pl.ds(e0 + i * CKE, CKE), :],
                             buf, sem)

        def consume(i, buf):
            pltpu.make_async_copy(ea_hbm.at[pl.ds(e0, CKE), :],
                                  buf, sem).wait()

            def repack(e, _):
                wide_v[e, pl.ds(0, DE)] = buf[e, pl.ds(0, DE)]
                return 0

            lax.fori_loop(0, CKE, repack, 0, unroll=False)
            pltpu.sync_copy(wide_v, acc_q.at[dstall.at[i]], add=True)

        start(0, eav0)

        def pair_body(g, _):
            i = 2 * g
            start(i + 1, eav1)
            consume(i, eav0)
            start(i + 2, eav0)
            consume(i + 1, eav1)
            return 0

        lax.fori_loop(0, NCHUNK // 2, pair_body, 0, unroll=False)
        consume(NCHUNK - 1, eav0)
        plsc.subcore_barrier()

        _copy_rows(lambda o, z: acc_q.at[pl.ds(r0 + o, z), :],
                   lambda o, z: q_out.at[c, pl.ds(r0 + o, z), :], CKE)

    return run(edge_attr, dst3)


_BM1 = 1000


def _tc_layer1(xs_p, q_p, Wn1, Wm1, We1, bn1, be1, bm1):
    def body(xs_ref, q_ref, wn1_ref, wm1_ref, we1_ref, bn1_ref,
             be1_ref, bm1_ref, out_ref):
        xs = xs_ref[0] + xs_ref[1]
        q = q_ref[0] + q_ref[1]
        ea = q[:, 0:DE]
        dg = q[:, DE:DE + 1]
        wm1 = wm1_ref[...]
        wt = wn1_ref[...] @ wm1
        we = we1_ref[...] @ wm1
        bt = (bn1_ref[...] + be1_ref[...]) @ wm1 + bm1_ref[...]
        sv = xs @ wt + ea @ we + dg * bt
        out_ref[...] = jnp.maximum(sv / jnp.clip(dg, 1.0, None), 0.0)

    grid = (N // _BM1,)
    return pl.pallas_call(
        body,
        grid=grid,
        in_specs=[
            pl.BlockSpec((NC, _BM1, DIN), lambda i: (0, i, 0)),
            pl.BlockSpec((NC, _BM1, DIN), lambda i: (0, i, 0)),
            pl.BlockSpec((DIN, DH), lambda i: (0, 0)),
            pl.BlockSpec((DH, DH), lambda i: (0, 0)),
            pl.BlockSpec((DE, DH), lambda i: (0, 0)),
            pl.BlockSpec((1, DH), lambda i: (0, 0)),
            pl.BlockSpec((1, DH), lambda i: (0, 0)),
            pl.BlockSpec((1, DH), lambda i: (0, 0)),
        ],
        out_specs=pl.BlockSpec((_BM1, DH), lambda i: (i, 0)),
        out_shape=jax.ShapeDtypeStruct((N, DH), jnp.float32),
    )(xs_p, q_p, Wn1, Wm1, We1, bn1, be1, bm1)


def _tc_z(s2_p, q_p, Wn2, bn2, We2, be2, Wm2, bm2):
    def body(s2_ref, q_ref, wn2_ref, bn2_ref, we2_ref, be2_ref,
             wm2_ref, bm2_ref, z_ref):
        s2 = s2_ref[0] + s2_ref[1]
        q = q_ref[0] + q_ref[1]
        ea = q[:, 0:DE]
        dg = q[:, DE:DE + 1]
        wm2 = wm2_ref[...]
        wn = wn2_ref[...] @ wm2
        we = we2_ref[...] @ wm2
        bt = (bn2_ref[...] + be2_ref[...]) @ wm2 + bm2_ref[...]
        sv = s2 @ wn + ea @ we + dg * bt
        z_ref[...] = sv / jnp.clip(dg, 1.0, None)

    grid = (N // _BM1,)
    return pl.pallas_call(
        body,
        grid=grid,
        in_specs=[
            pl.BlockSpec((NC, _BM1, DH), lambda i: (0, i, 0)),
            pl.BlockSpec((NC, _BM1, DIN), lambda i: (0, i, 0)),
            pl.BlockSpec((DH, DL), lambda i: (0, 0)),
            pl.BlockSpec((1, DL), lambda i: (0, 0)),
            pl.BlockSpec((DE, DL), lambda i: (0, 0)),
            pl.BlockSpec((1, DL), lambda i: (0, 0)),
            pl.BlockSpec((DL, DL), lambda i: (0, 0)),
            pl.BlockSpec((1, DL), lambda i: (0, 0)),
        ],
        out_specs=pl.BlockSpec((_BM1, DL), lambda i: (i, 0)),
        out_shape=jax.ShapeDtypeStruct((N, DL), jnp.float32),
    )(s2_p, q_p, Wn2, bn2, We2, be2, Wm2, bm2)


_BMA = 400


def _tc_adj(z):
    def body(zr_ref, zc_ref, out_ref):
        g = lax.dot_general(zr_ref[...], zc_ref[...],
                            (((1,), (1,)), ((), ())),
                            preferred_element_type=jnp.float32)
        out_ref[...] = jax.nn.sigmoid(g)

    grid = (N // _BMA,)
    return pl.pallas_call(
        body,
        grid=grid,
        in_specs=[
            pl.BlockSpec((_BMA, DL), lambda i: (i, 0)),
            pl.BlockSpec((N, DL), lambda i: (0, 0)),
        ],
        out_specs=pl.BlockSpec((_BMA, N), lambda i: (i, 0)),
        out_shape=jax.ShapeDtypeStruct((N, N), jnp.float32),
    )(z, z)


def kernel(x, edge_attr, edge_index, Wn1, bn1, We1, be1, Wm1, bm1,
           Wn2, bn2, We2, be2, Wm2, bm2):
    src_idx = edge_index[0]
    dst_idx = edge_index[1]
    dst3 = dst_idx.reshape(NW, NCHUNK, CK1)
    q_p = _sc_edge_passE(edge_attr, dst3)
    xs_p = _sc_gather_pass(x, src_idx, dst3, DIN)
    h1 = _tc_layer1(xs_p, q_p, Wn1, Wm1, We1,
                    bn1.reshape(1, DH), be1.reshape(1, DH),
                    bm1.reshape(1, DH))
    s2_p = _sc_gather_pass(h1, src_idx, dst3, DH)
    z = _tc_z(s2_p, q_p, Wn2, bn2.reshape(1, DL), We2,
              be2.reshape(1, DL), Wm2, bm2.reshape(1, DL))
    adj = _tc_adj(z)
    return (z, adj)


# speedup vs baseline: 4.7100x; 1.1194x over previous
import functools

import jax
import jax.numpy as jnp
from jax import lax
from jax.experimental import pallas as pl
from jax.experimental.pallas import tpu as pltpu
from jax.experimental.pallas import tpu_sc as plsc

N = 10000
E = 160000
DIN = 128
DE = 16
DH = 128
DL = 64

NC = 2
NS = 16
NW = NC * NS
CK1 = 40
EPW = E // NW
NPAD = 10240
RPT = NPAD // NS

_SC_MESH = dict(core_axis_name="c", subcore_axis_name="s",
                num_cores=NC, num_subcores=NS)


def _zero_vmem_2d(ref, rows, cols):
    zv = jnp.zeros((16,), jnp.float32)

    def body(i, _):
        for j in range(cols // 16):
            ref[i, pl.ds(j * 16, 16)] = zv
        return 0

    lax.fori_loop(0, rows, body, 0, unroll=False)


def _copy_rows(src_at, dst_at, chunk):
    for k in range(RPT // chunk):
        pltpu.sync_copy(src_at(k * chunk, chunk), dst_at(k * chunk, chunk))


NCHUNK = EPW // CK1


def _sc_gather_pass(table, src_idx, dst3, width):
    mesh = plsc.VectorSubcoreMesh(**_SC_MESH)

    @functools.partial(
        pl.kernel,
        out_type=jax.ShapeDtypeStruct((NC, NPAD, width), jnp.float32),
        mesh=mesh,
        scratch_types=[
            pltpu.VMEM_SHARED((NPAD, width), jnp.float32),
            pltpu.VMEM((CK1, width), jnp.float32),
            pltpu.VMEM((CK1, width), jnp.float32),
            pltpu.VMEM((CK1, width), jnp.float32),
            pltpu.VMEM((CK1, width), jnp.float32),
            pltpu.VMEM((EPW,), jnp.int32),
            pltpu.VMEM((NCHUNK, CK1), jnp.int32),
            pltpu.SemaphoreType.DMA,
        ],
    )
    def run(t_hbm, si_hbm, d2_hbm, acc_out,
            acc, rows0, rows1, rows2, rows3, srcall, dstall, sem):
        c = lax.axis_index("c")
        s = lax.axis_index("s")
        wid = s * NC + c

        _zero_vmem_2d(rows0, CK1, width)

        r0 = s * RPT
        _copy_rows(lambda o, z: rows0.at[pl.ds(0, z), :],
                   lambda o, z: acc.at[pl.ds(r0 + o, z), :], CK1)

        pltpu.sync_copy(si_hbm.at[pl.ds(wid * EPW, EPW)], srcall)
        pltpu.sync_copy(d2_hbm.at[wid], dstall)
        plsc.subcore_barrier()

        bufs = (rows0, rows1, rows2, rows3)

        def start(i, buf):
            pltpu.async_copy(t_hbm.at[srcall.at[pl.ds(i * CK1, CK1)]],
                             buf, sem)

        def drain(buf):
            pltpu.make_async_copy(t_hbm.at[srcall.at[pl.ds(0, CK1)]],
                                  buf, sem).wait()

        def scatter(i, buf):
            drain(buf)
            pltpu.sync_copy(buf, acc.at[dstall.at[i]], add=True)

        for b in range(4):
            start(b, bufs[b])

        def quad_body(g, _):
            i = 4 * g
            for b in range(4):
                scatter(i + b, bufs[b])
                start(i + 4 + b, bufs[b])
            return 0

        lax.fori_loop(0, (NCHUNK - 5) // 4, quad_body, 0, unroll=False)
        base = NCHUNK - 5
        for b in range(4):
            scatter(base + b, bufs[b])
        start(NCHUNK - 1, bufs[0])
        scatter(NCHUNK - 1, bufs[0])
        plsc.subcore_barrier()

        _copy_rows(lambda o, z: acc.at[pl.ds(r0 + o, z), :],
                   lambda o, z: acc_out.at[c, pl.ds(r0 + o, z), :], CK1)

    return run(table, src_idx, dst3)


CKE = 40


def _sc_edge_passE(edge_attr, dst3):
    mesh = plsc.VectorSubcoreMesh(**_SC_MESH)

    @functools.partial(
        pl.kernel,
        out_type=jax.ShapeDtypeStruct((NC, NPAD, DIN), jnp.float32),
        mesh=mesh,
        scratch_types=[
            pltpu.VMEM_SHARED((NPAD, DIN), jnp.float32),
            pltpu.VMEM((CKE, DIN), jnp.float32),
            pltpu.VMEM((CKE, DE), jnp.float32),
            pltpu.VMEM((CKE, DE), jnp.float32),
            pltpu.VMEM((NCHUNK, CKE), jnp.int32),
            pltpu.SemaphoreType.DMA,
        ],
    )
    def run(ea_hbm, d2_hbm, q_out, acc_q, wide_v, eav0, eav1, dstall, sem):
        c = lax.axis_index("c")
        s = lax.axis_index("s")
        wid = s * NC + c

        _zero_vmem_2d(wide_v, CKE, DIN)

        r0 = s * RPT
        _copy_rows(lambda o, z: wide_v.at[pl.ds(0, z), :],
                   lambda o, z: acc_q.at[pl.ds(r0 + o, z), :], CKE)

        one = jnp.full((16,), 1.0, jnp.float32)

        def initrow(i, _):
            wide_v[i, pl.ds(DE, 16)] = one
            return 0

        lax.fori_loop(0, CKE, initrow, 0, unroll=False)
        pltpu.sync_copy(d2_hbm.at[wid], dstall)
        plsc.subcore_barrier()

        e0 = wid * EPW

        def start(i, buf):
            pltpu.async_copy(ea_hbm.at[pl.ds(e0 + i * CKE, CKE), :],
                             buf, sem)

        def consume(i, buf):
            pltpu.make_async_copy(ea_hbm.at[pl.ds(e0, CKE), :],
                                  buf, sem).wait()

            def repack(e, _):
                wide_v[e, pl.ds(0, DE)] = buf[e, pl.ds(0, DE)]
                return 0

            lax.fori_loop(0, CKE, repack, 0, unroll=False)
            pltpu.sync_copy(wide_v, acc_q.at[dstall.at[i]], add=True)

        start(0, eav0)

        def pair_body(g, _):
            i = 2 * g
            start(i + 1, eav1)
            consume(i, eav0)
            start(i + 2, eav0)
            consume(i + 1, eav1)
            return 0

        lax.fori_loop(0, NCHUNK // 2, pair_body, 0, unroll=False)
        consume(NCHUNK - 1, eav0)
        plsc.subcore_barrier()

        _copy_rows(lambda o, z: acc_q.at[pl.ds(r0 + o, z), :],
                   lambda o, z: q_out.at[c, pl.ds(r0 + o, z), :], CKE)

    return run(edge_attr, dst3)


_BM1 = 1000


def _tc_layer1(xs_p, q_p, Wn1, Wm1, We1, bn1, be1, bm1):
    def body(xs_ref, q_ref, wn1_ref, wm1_ref, we1_ref, bn1_ref,
             be1_ref, bm1_ref, out_ref):
        xs = xs_ref[0] + xs_ref[1]
        q = q_ref[0] + q_ref[1]
        ea = q[:, 0:DE]
        dg = q[:, DE:DE + 1]
        wm1 = wm1_ref[...]
        wt = wn1_ref[...] @ wm1
        we = we1_ref[...] @ wm1
        bt = (bn1_ref[...] + be1_ref[...]) @ wm1 + bm1_ref[...]
        sv = xs @ wt + ea @ we + dg * bt
        out_ref[...] = jnp.maximum(sv / jnp.clip(dg, 1.0, None), 0.0)

    grid = (N // _BM1,)
    return pl.pallas_call(
        body,
        grid=grid,
        in_specs=[
            pl.BlockSpec((NC, _BM1, DIN), lambda i: (0, i, 0)),
            pl.BlockSpec((NC, _BM1, DIN), lambda i: (0, i, 0)),
            pl.BlockSpec((DIN, DH), lambda i: (0, 0)),
            pl.BlockSpec((DH, DH), lambda i: (0, 0)),
            pl.BlockSpec((DE, DH), lambda i: (0, 0)),
            pl.BlockSpec((1, DH), lambda i: (0, 0)),
            pl.BlockSpec((1, DH), lambda i: (0, 0)),
            pl.BlockSpec((1, DH), lambda i: (0, 0)),
        ],
        out_specs=pl.BlockSpec((_BM1, DH), lambda i: (i, 0)),
        out_shape=jax.ShapeDtypeStruct((N, DH), jnp.float32),
    )(xs_p, q_p, Wn1, Wm1, We1, bn1, be1, bm1)


def _tc_z(s2_p, q_p, Wn2, bn2, We2, be2, Wm2, bm2):
    def body(s2_ref, q_ref, wn2_ref, bn2_ref, we2_ref, be2_ref,
             wm2_ref, bm2_ref, z_ref):
        s2 = s2_ref[0] + s2_ref[1]
        q = q_ref[0] + q_ref[1]
        ea = q[:, 0:DE]
        dg = q[:, DE:DE + 1]
        wm2 = wm2_ref[...]
        wn = wn2_ref[...] @ wm2
        we = we2_ref[...] @ wm2
        bt = (bn2_ref[...] + be2_ref[...]) @ wm2 + bm2_ref[...]
        sv = s2 @ wn + ea @ we + dg * bt
        z_ref[...] = sv / jnp.clip(dg, 1.0, None)

    grid = (N // _BM1,)
    return pl.pallas_call(
        body,
        grid=grid,
        in_specs=[
            pl.BlockSpec((NC, _BM1, DH), lambda i: (0, i, 0)),
            pl.BlockSpec((NC, _BM1, DIN), lambda i: (0, i, 0)),
            pl.BlockSpec((DH, DL), lambda i: (0, 0)),
            pl.BlockSpec((1, DL), lambda i: (0, 0)),
            pl.BlockSpec((DE, DL), lambda i: (0, 0)),
            pl.BlockSpec((1, DL), lambda i: (0, 0)),
            pl.BlockSpec((DL, DL), lambda i: (0, 0)),
            pl.BlockSpec((1, DL), lambda i: (0, 0)),
        ],
        out_specs=pl.BlockSpec((_BM1, DL), lambda i: (i, 0)),
        out_shape=jax.ShapeDtypeStruct((N, DL), jnp.float32),
    )(s2_p, q_p, Wn2, bn2, We2, be2, Wm2, bm2)


_BMA = 400


def _tc_adj(z):
    def body(zr_ref, zc_ref, out_ref):
        g = lax.dot_general(zr_ref[...], zc_ref[...],
                            (((1,), (1,)), ((), ())),
                            preferred_element_type=jnp.float32)
        out_ref[...] = jax.nn.sigmoid(g)

    grid = (N // _BMA,)
    return pl.pallas_call(
        body,
        grid=grid,
        in_specs=[
            pl.BlockSpec((_BMA, DL), lambda i: (i, 0)),
            pl.BlockSpec((N, DL), lambda i: (0, 0)),
        ],
        out_specs=pl.BlockSpec((_BMA, N), lambda i: (i, 0)),
        out_shape=jax.ShapeDtypeStruct((N, N), jnp.float32),
    )(z, z)


def kernel(x, edge_attr, edge_index, Wn1, bn1, We1, be1, Wm1, bm1,
           Wn2, bn2, We2, be2, Wm2, bm2):
    src_idx = edge_index[0]
    dst_idx = edge_index[1]
    dst3 = dst_idx.reshape(NW, NCHUNK, CK1)
    q_p = _sc_edge_passE(edge_attr, dst3)
    xs_p = _sc_gather_pass(x, src_idx, dst3, DIN)
    h1 = _tc_layer1(xs_p, q_p, Wn1, Wm1, We1,
                    bn1.reshape(1, DH), be1.reshape(1, DH),
                    bm1.reshape(1, DH))
    s2_p = _sc_gather_pass(h1, src_idx, dst3, DH)
    z = _tc_z(s2_p, q_p, Wn2, bn2.reshape(1, DL), We2,
              be2.reshape(1, DL), Wm2, bm2.reshape(1, DL))
    adj = _tc_adj(z)
    return (z, adj)
